# baseline (speedup 1.0000x reference)
import functools

import jax
import jax.numpy as jnp
from jax import lax
from jax.experimental import pallas as pl
from jax.experimental.pallas import tpu as pltpu
from jax.experimental.pallas import tpu_sc as plsc

_NP = 10
_N = 100
_D = 16
_L = 16
_NC = 2
_NS = 16
_NW = _NC * _NS
_BLK = 128
_SC_ROWS = 512


def _splat_i(v):
    return jnp.full((_L,), v, jnp.int32)


def _splat_f(v):
    return jnp.full((_L,), v, jnp.float32)


def _pw(tab_v, x, w):
    xp = x * 16.0
    ip = xp.astype(jnp.int32)
    fp = xp - ip.astype(jnp.float32)
    base = _splat_i(w * 17)
    c1 = plsc.load_gather(tab_v, [jnp.minimum(ip, 16) + base])
    n2 = plsc.load_gather(tab_v, [jnp.minimum(ip + 1, 16) + base + 136])
    return c1 + fp * n2, ip, fp


def _pw2(tab_v, ip, fp, w):
    base = _splat_i(w * 17)
    c1 = plsc.load_gather(tab_v, [jnp.minimum(ip, 16) + base])
    n2 = plsc.load_gather(tab_v, [jnp.minimum(ip + 1, 16) + base + 136])
    return c1 + fp * n2


def _sc_body(gpw, att_hbm, boxes_hbm, tab_hbm, par_hbm, out_hbm,
             att_v, boxes_v, tab_v, par_v, at_v, out_v):
    wid = lax.axis_index("s") * _NC + lax.axis_index("c")
    pltpu.sync_copy(tab_hbm, tab_v)
    pltpu.sync_copy(par_hbm, par_v)
    factor = par_v[...]
    lane = lax.iota(jnp.int32, _L)
    ninf = _splat_f(-jnp.inf)

    def group_body(gi, carry):
        g = wid * gpw + gi
        pltpu.sync_copy(att_hbm.at[pl.ds(g * _L, _L)], att_v)
        pltpu.sync_copy(boxes_hbm.at[pl.ds(g * _L, _L)], boxes_v)

        cm = []
        for c in range(10):
            m = plsc.load_gather(att_v, [lane, _splat_i(10 * c)])
            for r in range(1, 10):
                m = jnp.maximum(
                    m, plsc.load_gather(att_v, [lane, _splat_i(10 * c + r)]))
            cm.append(m)

        att_top, nsel = [], []
        for _ in range(_NP):
            m = cm[0]
            for c in range(1, 10):
                m = jnp.maximum(m, cm[c])
            cstar = _splat_i(9)
            for c in range(8, -1, -1):
                cstar = jnp.where(cm[c] == m, c, cstar)
            base = cstar * 10
            vs = [None] * 10
            nstar = base + 9
            for r in range(9, -1, -1):
                vs[r] = plsc.load_gather(att_v, [lane, base + r])
                nstar = jnp.where(vs[r] == m, base + r, nstar)
            att_top.append(m)
            nsel.append(nstar)
            plsc.store_scatter(att_v, [lane, nstar], ninf)
            nm = ninf
            for r in range(10):
                nm = jnp.maximum(nm, jnp.where(base + r == nstar, ninf, vs[r]))
            for c in range(10):
                cm[c] = jnp.where(cstar == c, nm, cm[c])

        att10, y0s, x0s, y1s, x1s, areas = [], [], [], [], [], []
        for k in range(_NP):
            n_k = nsel[k]
            y0 = plsc.load_gather(boxes_v, [lane, n_k])
            x0 = plsc.load_gather(boxes_v, [lane, n_k + 100])
            y1 = plsc.load_gather(boxes_v, [lane, n_k + 200])
            x1 = plsc.load_gather(boxes_v, [lane, n_k + 300])
            y0s.append(y0); x0s.append(x0); y1s.append(y1); x1s.append(x1)
            areas.append(jnp.maximum(y1 - y0, 0.0) * jnp.maximum(x1 - x0, 0.0))
            x = att_top[k]
            e = jnp.exp(-jnp.abs(x))
            r = 1.0 / (1.0 + e)
            att10.append(jnp.where(x >= 0.0, r, e * r))

        s_acc = [_splat_f(0.0) for _ in range(_NP)]
        pd_sum = _splat_f(0.0)
        pid = 0
        for i in range(_NP):
            for j in range(i, _NP):
                A = att10[i] * att10[j]
                mny = jnp.maximum(y0s[i], y0s[j])
                mnx = jnp.maximum(x0s[i], x0s[j])
                mxy = jnp.minimum(y1s[i], y1s[j])
                mxx = jnp.minimum(x1s[i], x1s[j])
                ia = (jnp.maximum(mxy - mny, 0.0)
                      * jnp.maximum(mxx - mnx, 0.0))
                Dm = 1.0 - ia / (areas[i] + areas[j] - ia + 1e-12)
                plA0, _, _ = _pw(tab_v, A, 0)
                plD1, ipD, fpD = _pw(tab_v, Dm, 1)
                plD6 = _pw2(tab_v, ipD, fpD, 6)
                ad = 1.0 - jnp.abs(att10[i] - att10[j])
                sim, _, _ = _pw(tab_v, ad, 2)
                at_v[pid, :] = plA0 * plD1
                s_acc[i] = s_acc[i] + sim
                if i != j:
                    s_acc[j] = s_acc[j] + sim
                pd = jnp.abs(plD6 - 0.5)
                pd_sum = pd_sum + (pd if i == j else pd + pd)
                pid += 1

        inv_s = []
        for i in range(_NP):
            s_acc[i] = s_acc[i] * factor
            inv_s.append(1.0 / s_acc[i])

        modE = _splat_f(0.0)
        pid = 0
        for i in range(_NP):
            for j in range(i, _NP):
                sc = at_v[pid, :] * (inv_s[i] * inv_s[j])
                modE = modE + (sc if i == j else sc + sc)
                pid += 1
        for i in range(_NP):
            cn, _, _ = _pw(tab_v, att10[i] * att10[i], 0)
            modE = modE + cn * inv_s[i]

        a = modE + 1e-20
        zi = jnp.int32(0x5F3759DF) - (plsc.bitcast(a, jnp.int32) >> 1)
        y = plsc.bitcast(zi, jnp.float32)
        for _ in range(3):
            y = y * (1.5 - 0.5 * a * y * y)
        cval = a * y

        cc = jnp.clip(cval, 0.0, 10.0)
        ip = cc.astype(jnp.int32)
        fp = cc - ip.astype(jnp.float32)
        ip1 = jnp.minimum(ip + 1, _NP)
        pa_sum = _splat_f(0.0)
        for i in range(_NP):
            pl5, _, _ = _pw(tab_v, att10[i], 5)
            pa_sum = pa_sum + jnp.abs(pl5 - 0.5)
        gate, _, _ = _pw(tab_v, pa_sum / 10.0 + pd_sum / 100.0, 7)
        one = _splat_f(1.0)
        zero = _splat_f(0.0)
        obase = lane * (_NP + 1)
        for t in range(_NP + 1):
            o_t = (jnp.where(ip == t, one - fp, zero)
                   + jnp.where(ip1 == t, fp, zero))
            plsc.store_scatter(out_v, [obase + t], gate * o_t)
        pltpu.sync_copy(out_v, out_hbm.at[pl.ds(g * _L * (_NP + 1),
                                                _L * (_NP + 1))])
        return carry

    lax.fori_loop(0, gpw, group_body, 0)


def _pw_multi(xp, coef_ref, rows):
    res = [coef_ref[r, 0] * (xp + 1.0) for r in rows]
    for s in range(_D):
        t = xp if s == 0 else jnp.maximum(xp - float(s), 0.0)
        for j, r in enumerate(rows):
            res[j] = res[j] + coef_ref[r, s + 1] * t
    return res


def _tc_block(att_ref, boxes_ref, coef_ref, fac_ref, out_ref):
    att_all = att_ref[...]
    iota_n = jax.lax.broadcasted_iota(jnp.int32, (_N, _BLK), 0)

    work = att_all
    vals, coords = [], [[] for _ in range(4)]
    for _ in range(_NP):
        m = jnp.max(work, axis=0, keepdims=True)
        cand = jnp.where(work == m, iota_n, _N)
        sel = jnp.min(cand, axis=0, keepdims=True)
        onehot = iota_n == sel
        vals.append(m)
        ohf = onehot.astype(jnp.float32)
        for c in range(4):
            coords[c].append(
                jnp.sum(boxes_ref[c] * ohf, axis=0, keepdims=True))
        work = jnp.where(onehot, -jnp.inf, work)

    att_top = jnp.concatenate(vals, axis=0)
    y0, x0, y1, x1 = (jnp.concatenate(cs, axis=0) for cs in coords)

    e = jnp.exp(-jnp.abs(att_top))
    att = jnp.where(att_top >= 0.0, 1.0 / (1.0 + e), e / (1.0 + e))

    ai = att[:, None, :]
    aj = att[None, :, :]
    A = ai * aj

    h = jnp.maximum(y1 - y0, 0.0)
    w = jnp.maximum(x1 - x0, 0.0)
    areas = h * w
    mny = jnp.maximum(y0[:, None, :], y0[None, :, :])
    mnx = jnp.maximum(x0[:, None, :], x0[None, :, :])
    mxy = jnp.minimum(y1[:, None, :], y1[None, :, :])
    mxx = jnp.minimum(x1[:, None, :], x1[None, :, :])
    ia = jnp.maximum(mxy - mny, 0.0) * jnp.maximum(mxx - mnx, 0.0)
    iou = ia / (areas[:, None, :] + areas[None, :, :] - ia + 1e-12)
    Dm = 1.0 - iou

    (plA0,) = _pw_multi(A * 16.0, coef_ref, [0])
    plD1, plD6 = _pw_multi(Dm * 16.0, coef_ref, [1, 6])
    att_diff = jnp.abs(ai - aj)
    (sim,) = _pw_multi((1.0 - att_diff) * 16.0, coef_ref, [2])

    A_tilde = plA0 * plD1
    s_i = fac_ref[0, 0] * jnp.sum(sim, axis=1)
    score = A_tilde / (s_i[:, None, :] * s_i[None, :, :])
    (corr_num,) = _pw_multi(att * att * 16.0, coef_ref, [0])
    corr = corr_num / s_i
    mod_E = (jnp.sum(jnp.sum(score, axis=1), axis=0, keepdims=True)
             + jnp.sum(corr, axis=0, keepdims=True))
    c = jnp.sqrt(mod_E + 1e-20)

    cc = jnp.clip(c, 0.0, float(_NP))
    ip = cc.astype(jnp.int32)
    fp = cc - jnp.trunc(cc)
    iota11 = jax.lax.broadcasted_iota(jnp.int32, (_NP + 1, _BLK), 0)
    left = (iota11 == ip).astype(jnp.float32)
    right = (iota11 == jnp.minimum(ip + 1, _NP)).astype(jnp.float32)
    o = (1.0 - fp) * left + fp * right

    (pl5,) = _pw_multi(att * 16.0, coef_ref, [5])
    p_a = jnp.abs(pl5 - 0.5)
    pam = jnp.sum(p_a, axis=0, keepdims=True) / float(_NP)
    p_d = jnp.abs(plD6 - 0.5)
    pdm = jnp.sum(jnp.sum(p_d, axis=1) / float(_NP),
                  axis=0, keepdims=True) / float(_NP)
    (gate,) = _pw_multi((pam + pdm) * 16.0, coef_ref, [7])
    out_ref[...] = gate * o


@jax.jit
def kernel(boxes, attention, Ws):
    B = attention.shape[0]
    S = _SC_ROWS
    Btc = B - S

    aw = jnp.abs(Ws)
    nw = aw / jnp.sum(aw, axis=1, keepdims=True)
    cs = jnp.cumsum(nw, axis=1)
    fac = cs[2, 16] ** _NP

    if S:
        boxes_flat = boxes[:S].reshape(S, 4 * _N)
        tab = jnp.concatenate([cs.reshape(-1), nw.reshape(-1)])
        par = jnp.full((16,), fac, jnp.float32)
        mesh = plsc.VectorSubcoreMesh(core_axis_name="c",
                                      subcore_axis_name="s")
        sc_fn = pl.kernel(
            functools.partial(_sc_body, S // _L // _NW),
            out_type=jax.ShapeDtypeStruct((S * (_NP + 1),), jnp.float32),
            mesh=mesh,
            scratch_types=[
                pltpu.VMEM((_L, _N), jnp.float32),
                pltpu.VMEM((_L, 4 * _N), jnp.float32),
                pltpu.VMEM((272,), jnp.float32),
                pltpu.VMEM((16,), jnp.float32),
                pltpu.VMEM((56, _L), jnp.float32),
                pltpu.VMEM((_L * (_NP + 1),), jnp.float32),
            ],
            compiler_params=pltpu.CompilerParams(needs_layout_passes=False),
        )
        sc_out = sc_fn(attention[:S], boxes_flat, tab, par)

    att_t = attention[S:].T
    boxes_t = boxes[S:].transpose(1, 2, 0)
    coef = jnp.concatenate([nw[:, :1], nw[:, 1:] - nw[:, :-1]], axis=1)
    fac2 = fac.reshape(1, 1)
    tc_out = pl.pallas_call(
        _tc_block,
        grid=(Btc // _BLK,),
        in_specs=[
            pl.BlockSpec((_N, _BLK), lambda i: (0, i)),
            pl.BlockSpec((4, _N, _BLK), lambda i: (0, 0, i)),
            pl.BlockSpec(memory_space=pltpu.SMEM),
            pl.BlockSpec(memory_space=pltpu.SMEM),
        ],
        out_specs=pl.BlockSpec((_NP + 1, _BLK), lambda i: (0, i)),
        out_shape=jax.ShapeDtypeStruct((_NP + 1, Btc), jnp.float32),
    )(att_t, boxes_t, coef, fac2)

    if not S:
        return tc_out.T
    return jnp.concatenate([sc_out.reshape(S, _NP + 1), tc_out.T], axis=0)


# speedup vs baseline: 1.0350x; 1.0350x over previous
import functools

import jax
import jax.numpy as jnp
from jax import lax
from jax.experimental import pallas as pl
from jax.experimental.pallas import tpu as pltpu
from jax.experimental.pallas import tpu_sc as plsc

_NP = 10
_N = 100
_D = 16
_L = 16
_NC = 2
_NS = 16
_NW = _NC * _NS
_BLK = 128
_SC_ROWS = 1024


def _splat_i(v):
    return jnp.full((_L,), v, jnp.int32)


def _splat_f(v):
    return jnp.full((_L,), v, jnp.float32)


def _pw(tab_v, x, w):
    xp = x * 16.0
    ip = xp.astype(jnp.int32)
    fp = xp - ip.astype(jnp.float32)
    base = _splat_i(w * 17)
    c1 = plsc.load_gather(tab_v, [jnp.minimum(ip, 16) + base])
    n2 = plsc.load_gather(tab_v, [jnp.minimum(ip + 1, 16) + base + 136])
    return c1 + fp * n2, ip, fp


def _pw2(tab_v, ip, fp, w):
    base = _splat_i(w * 17)
    c1 = plsc.load_gather(tab_v, [jnp.minimum(ip, 16) + base])
    n2 = plsc.load_gather(tab_v, [jnp.minimum(ip + 1, 16) + base + 136])
    return c1 + fp * n2


def _sc_body(gpw, att_hbm, boxes_hbm, tab_hbm, par_hbm, out_hbm,
             att_v, boxes_v, tab_v, par_v, at_v, out_v):
    wid = lax.axis_index("s") * _NC + lax.axis_index("c")
    pltpu.sync_copy(tab_hbm, tab_v)
    pltpu.sync_copy(par_hbm, par_v)
    factor = par_v[...]
    lane = lax.iota(jnp.int32, _L)
    ninf = _splat_f(-jnp.inf)

    def group_body(gi, carry):
        g = wid * gpw + gi
        pltpu.sync_copy(att_hbm.at[pl.ds(g * _L, _L)], att_v)
        pltpu.sync_copy(boxes_hbm.at[pl.ds(g * _L, _L)], boxes_v)

        cm = []
        for c in range(10):
            m = plsc.load_gather(att_v, [lane, _splat_i(10 * c)])
            for r in range(1, 10):
                m = jnp.maximum(
                    m, plsc.load_gather(att_v, [lane, _splat_i(10 * c + r)]))
            cm.append(m)

        att_top, nsel = [], []
        for _ in range(_NP):
            m = cm[0]
            for c in range(1, 10):
                m = jnp.maximum(m, cm[c])
            cstar = _splat_i(9)
            for c in range(8, -1, -1):
                cstar = jnp.where(cm[c] == m, c, cstar)
            base = cstar * 10
            vs = [None] * 10
            nstar = base + 9
            for r in range(9, -1, -1):
                vs[r] = plsc.load_gather(att_v, [lane, base + r])
                nstar = jnp.where(vs[r] == m, base + r, nstar)
            att_top.append(m)
            nsel.append(nstar)
            plsc.store_scatter(att_v, [lane, nstar], ninf)
            nm = ninf
            for r in range(10):
                nm = jnp.maximum(nm, jnp.where(base + r == nstar, ninf, vs[r]))
            for c in range(10):
                cm[c] = jnp.where(cstar == c, nm, cm[c])

        att10, y0s, x0s, y1s, x1s, areas = [], [], [], [], [], []
        for k in range(_NP):
            n_k = nsel[k]
            y0 = plsc.load_gather(boxes_v, [lane, n_k])
            x0 = plsc.load_gather(boxes_v, [lane, n_k + 100])
            y1 = plsc.load_gather(boxes_v, [lane, n_k + 200])
            x1 = plsc.load_gather(boxes_v, [lane, n_k + 300])
            y0s.append(y0); x0s.append(x0); y1s.append(y1); x1s.append(x1)
            areas.append(jnp.maximum(y1 - y0, 0.0) * jnp.maximum(x1 - x0, 0.0))
            x = att_top[k]
            e = jnp.exp(-jnp.abs(x))
            r = 1.0 / (1.0 + e)
            att10.append(jnp.where(x >= 0.0, r, e * r))

        s_acc = [_splat_f(0.0) for _ in range(_NP)]
        pd_sum = _splat_f(0.0)
        pid = 0
        for i in range(_NP):
            for j in range(i, _NP):
                A = att10[i] * att10[j]
                mny = jnp.maximum(y0s[i], y0s[j])
                mnx = jnp.maximum(x0s[i], x0s[j])
                mxy = jnp.minimum(y1s[i], y1s[j])
                mxx = jnp.minimum(x1s[i], x1s[j])
                ia = (jnp.maximum(mxy - mny, 0.0)
                      * jnp.maximum(mxx - mnx, 0.0))
                Dm = 1.0 - ia / (areas[i] + areas[j] - ia + 1e-12)
                plA0, _, _ = _pw(tab_v, A, 0)
                plD1, ipD, fpD = _pw(tab_v, Dm, 1)
                plD6 = _pw2(tab_v, ipD, fpD, 6)
                ad = 1.0 - jnp.abs(att10[i] - att10[j])
                sim, _, _ = _pw(tab_v, ad, 2)
                at_v[pid, :] = plA0 * plD1
                s_acc[i] = s_acc[i] + sim
                if i != j:
                    s_acc[j] = s_acc[j] + sim
                pd = jnp.abs(plD6 - 0.5)
                pd_sum = pd_sum + (pd if i == j else pd + pd)
                pid += 1

        inv_s = []
        for i in range(_NP):
            s_acc[i] = s_acc[i] * factor
            inv_s.append(1.0 / s_acc[i])

        modE = _splat_f(0.0)
        pid = 0
        for i in range(_NP):
            for j in range(i, _NP):
                sc = at_v[pid, :] * (inv_s[i] * inv_s[j])
                modE = modE + (sc if i == j else sc + sc)
                pid += 1
        for i in range(_NP):
            cn, _, _ = _pw(tab_v, att10[i] * att10[i], 0)
            modE = modE + cn * inv_s[i]

        a = modE + 1e-20
        zi = jnp.int32(0x5F3759DF) - (plsc.bitcast(a, jnp.int32) >> 1)
        y = plsc.bitcast(zi, jnp.float32)
        for _ in range(3):
            y = y * (1.5 - 0.5 * a * y * y)
        cval = a * y

        cc = jnp.clip(cval, 0.0, 10.0)
        ip = cc.astype(jnp.int32)
        fp = cc - ip.astype(jnp.float32)
        ip1 = jnp.minimum(ip + 1, _NP)
        pa_sum = _splat_f(0.0)
        for i in range(_NP):
            pl5, _, _ = _pw(tab_v, att10[i], 5)
            pa_sum = pa_sum + jnp.abs(pl5 - 0.5)
        gate, _, _ = _pw(tab_v, pa_sum / 10.0 + pd_sum / 100.0, 7)
        one = _splat_f(1.0)
        zero = _splat_f(0.0)
        obase = lane * (_NP + 1)
        for t in range(_NP + 1):
            o_t = (jnp.where(ip == t, one - fp, zero)
                   + jnp.where(ip1 == t, fp, zero))
            plsc.store_scatter(out_v, [obase + t], gate * o_t)
        pltpu.sync_copy(out_v, out_hbm.at[pl.ds(g * _L * (_NP + 1),
                                                _L * (_NP + 1))])
        return carry

    lax.fori_loop(0, gpw, group_body, 0)


def _pw_multi(xp, coef_ref, rows):
    res = [coef_ref[r, 0] * (xp + 1.0) for r in rows]
    for s in range(_D):
        t = xp if s == 0 else jnp.maximum(xp - float(s), 0.0)
        for j, r in enumerate(rows):
            res[j] = res[j] + coef_ref[r, s + 1] * t
    return res


def _tc_block(att_ref, boxes_ref, coef_ref, fac_ref, out_ref):
    att_all = att_ref[...]
    iota_n = jax.lax.broadcasted_iota(jnp.int32, (_N, _BLK), 0)

    work = att_all
    vals, coords = [], [[] for _ in range(4)]
    for _ in range(_NP):
        m = jnp.max(work, axis=0, keepdims=True)
        cand = jnp.where(work == m, iota_n, _N)
        sel = jnp.min(cand, axis=0, keepdims=True)
        onehot = iota_n == sel
        vals.append(m)
        ohf = onehot.astype(jnp.float32)
        for c in range(4):
            coords[c].append(
                jnp.sum(boxes_ref[c] * ohf, axis=0, keepdims=True))
        work = jnp.where(onehot, -jnp.inf, work)

    att_top = jnp.concatenate(vals, axis=0)
    y0, x0, y1, x1 = (jnp.concatenate(cs, axis=0) for cs in coords)

    e = jnp.exp(-jnp.abs(att_top))
    att = jnp.where(att_top >= 0.0, 1.0 / (1.0 + e), e / (1.0 + e))

    ai = att[:, None, :]
    aj = att[None, :, :]
    A = ai * aj

    h = jnp.maximum(y1 - y0, 0.0)
    w = jnp.maximum(x1 - x0, 0.0)
    areas = h * w
    mny = jnp.maximum(y0[:, None, :], y0[None, :, :])
    mnx = jnp.maximum(x0[:, None, :], x0[None, :, :])
    mxy = jnp.minimum(y1[:, None, :], y1[None, :, :])
    mxx = jnp.minimum(x1[:, None, :], x1[None, :, :])
    ia = jnp.maximum(mxy - mny, 0.0) * jnp.maximum(mxx - mnx, 0.0)
    iou = ia / (areas[:, None, :] + areas[None, :, :] - ia + 1e-12)
    Dm = 1.0 - iou

    (plA0,) = _pw_multi(A * 16.0, coef_ref, [0])
    plD1, plD6 = _pw_multi(Dm * 16.0, coef_ref, [1, 6])
    att_diff = jnp.abs(ai - aj)
    (sim,) = _pw_multi((1.0 - att_diff) * 16.0, coef_ref, [2])

    A_tilde = plA0 * plD1
    s_i = fac_ref[0, 0] * jnp.sum(sim, axis=1)
    score = A_tilde / (s_i[:, None, :] * s_i[None, :, :])
    (corr_num,) = _pw_multi(att * att * 16.0, coef_ref, [0])
    corr = corr_num / s_i
    mod_E = (jnp.sum(jnp.sum(score, axis=1), axis=0, keepdims=True)
             + jnp.sum(corr, axis=0, keepdims=True))
    c = jnp.sqrt(mod_E + 1e-20)

    cc = jnp.clip(c, 0.0, float(_NP))
    ip = cc.astype(jnp.int32)
    fp = cc - jnp.trunc(cc)
    iota11 = jax.lax.broadcasted_iota(jnp.int32, (_NP + 1, _BLK), 0)
    left = (iota11 == ip).astype(jnp.float32)
    right = (iota11 == jnp.minimum(ip + 1, _NP)).astype(jnp.float32)
    o = (1.0 - fp) * left + fp * right

    (pl5,) = _pw_multi(att * 16.0, coef_ref, [5])
    p_a = jnp.abs(pl5 - 0.5)
    pam = jnp.sum(p_a, axis=0, keepdims=True) / float(_NP)
    p_d = jnp.abs(plD6 - 0.5)
    pdm = jnp.sum(jnp.sum(p_d, axis=1) / float(_NP),
                  axis=0, keepdims=True) / float(_NP)
    (gate,) = _pw_multi((pam + pdm) * 16.0, coef_ref, [7])
    out_ref[...] = gate * o


@jax.jit
def kernel(boxes, attention, Ws):
    B = attention.shape[0]
    S = _SC_ROWS
    Btc = B - S

    aw = jnp.abs(Ws)
    nw = aw / jnp.sum(aw, axis=1, keepdims=True)
    cs = jnp.cumsum(nw, axis=1)
    fac = cs[2, 16] ** _NP

    if S:
        boxes_flat = boxes[:S].reshape(S, 4 * _N)
        tab = jnp.concatenate([cs.reshape(-1), nw.reshape(-1)])
        par = jnp.full((16,), fac, jnp.float32)
        mesh = plsc.VectorSubcoreMesh(core_axis_name="c",
                                      subcore_axis_name="s")
        sc_fn = pl.kernel(
            functools.partial(_sc_body, S // _L // _NW),
            out_type=jax.ShapeDtypeStruct((S * (_NP + 1),), jnp.float32),
            mesh=mesh,
            scratch_types=[
                pltpu.VMEM((_L, _N), jnp.float32),
                pltpu.VMEM((_L, 4 * _N), jnp.float32),
                pltpu.VMEM((272,), jnp.float32),
                pltpu.VMEM((16,), jnp.float32),
                pltpu.VMEM((56, _L), jnp.float32),
                pltpu.VMEM((_L * (_NP + 1),), jnp.float32),
            ],
            compiler_params=pltpu.CompilerParams(needs_layout_passes=False, use_tc_tiling_on_sc=True),
        )
        sc_out = sc_fn(attention[:S], boxes_flat, tab, par)

    att_t = attention[S:].T
    boxes_t = boxes[S:].transpose(1, 2, 0)
    coef = jnp.concatenate([nw[:, :1], nw[:, 1:] - nw[:, :-1]], axis=1)
    fac2 = fac.reshape(1, 1)
    tc_out = pl.pallas_call(
        _tc_block,
        grid=(Btc // _BLK,),
        in_specs=[
            pl.BlockSpec((_N, _BLK), lambda i: (0, i)),
            pl.BlockSpec((4, _N, _BLK), lambda i: (0, 0, i)),
            pl.BlockSpec(memory_space=pltpu.SMEM),
            pl.BlockSpec(memory_space=pltpu.SMEM),
        ],
        out_specs=pl.BlockSpec((_NP + 1, _BLK), lambda i: (0, i)),
        out_shape=jax.ShapeDtypeStruct((_NP + 1, Btc), jnp.float32),
    )(att_t, boxes_t, coef, fac2)

    if not S:
        return tc_out.T
    return jnp.concatenate([sc_out.reshape(S, _NP + 1), tc_out.T], axis=0)


# speedup vs baseline: 1.1090x; 1.0715x over previous
import functools

import jax
import jax.numpy as jnp
from jax import lax
from jax.experimental import pallas as pl
from jax.experimental.pallas import tpu as pltpu
from jax.experimental.pallas import tpu_sc as plsc

_NP = 10
_N = 100
_D = 16
_L = 16
_NC = 2
_NS = 16
_NW = _NC * _NS
_BLK = 128
_SC_ROWS = 1536


def _splat_i(v):
    return jnp.full((_L,), v, jnp.int32)


def _splat_f(v):
    return jnp.full((_L,), v, jnp.float32)


def _pw(tab_v, x, w):
    xp = x * 16.0
    ip = xp.astype(jnp.int32)
    fp = xp - ip.astype(jnp.float32)
    base = _splat_i(w * 17)
    c1 = plsc.load_gather(tab_v, [jnp.minimum(ip, 16) + base])
    n2 = plsc.load_gather(tab_v, [jnp.minimum(ip + 1, 16) + base + 136])
    return c1 + fp * n2, ip, fp


def _pw2(tab_v, ip, fp, w):
    base = _splat_i(w * 17)
    c1 = plsc.load_gather(tab_v, [jnp.minimum(ip, 16) + base])
    n2 = plsc.load_gather(tab_v, [jnp.minimum(ip + 1, 16) + base + 136])
    return c1 + fp * n2


def _sc_body(gpw, att_hbm, boxes_hbm, tab_hbm, par_hbm, out_hbm,
             att_v, boxes_v, tab_v, par_v, at_v, out_v):
    wid = lax.axis_index("s") * _NC + lax.axis_index("c")
    pltpu.sync_copy(tab_hbm, tab_v)
    pltpu.sync_copy(par_hbm, par_v)
    factor = par_v[...]
    lane = lax.iota(jnp.int32, _L)
    ninf = _splat_f(-jnp.inf)

    def group_body(gi, carry):
        g = wid * gpw + gi
        pltpu.sync_copy(att_hbm.at[pl.ds(g * _L, _L)], att_v)
        pltpu.sync_copy(boxes_hbm.at[pl.ds(g * _L, _L)], boxes_v)

        cm = []
        for c in range(10):
            m = plsc.load_gather(att_v, [lane, _splat_i(10 * c)])
            for r in range(1, 10):
                m = jnp.maximum(
                    m, plsc.load_gather(att_v, [lane, _splat_i(10 * c + r)]))
            cm.append(m)

        att_top, nsel = [], []
        for _ in range(_NP):
            m = cm[0]
            for c in range(1, 10):
                m = jnp.maximum(m, cm[c])
            cstar = _splat_i(9)
            for c in range(8, -1, -1):
                cstar = jnp.where(cm[c] == m, c, cstar)
            base = cstar * 10
            vs = [None] * 10
            nstar = base + 9
            for r in range(9, -1, -1):
                vs[r] = plsc.load_gather(att_v, [lane, base + r])
                nstar = jnp.where(vs[r] == m, base + r, nstar)
            att_top.append(m)
            nsel.append(nstar)
            plsc.store_scatter(att_v, [lane, nstar], ninf)
            nm = ninf
            for r in range(10):
                nm = jnp.maximum(nm, jnp.where(base + r == nstar, ninf, vs[r]))
            for c in range(10):
                cm[c] = jnp.where(cstar == c, nm, cm[c])

        att10, y0s, x0s, y1s, x1s, areas = [], [], [], [], [], []
        for k in range(_NP):
            n_k = nsel[k]
            y0 = plsc.load_gather(boxes_v, [lane, n_k])
            x0 = plsc.load_gather(boxes_v, [lane, n_k + 100])
            y1 = plsc.load_gather(boxes_v, [lane, n_k + 200])
            x1 = plsc.load_gather(boxes_v, [lane, n_k + 300])
            y0s.append(y0); x0s.append(x0); y1s.append(y1); x1s.append(x1)
            areas.append(jnp.maximum(y1 - y0, 0.0) * jnp.maximum(x1 - x0, 0.0))
            x = att_top[k]
            e = jnp.exp(-jnp.abs(x))
            r = 1.0 / (1.0 + e)
            att10.append(jnp.where(x >= 0.0, r, e * r))

        s_acc = [_splat_f(0.0) for _ in range(_NP)]
        pd_sum = _splat_f(0.0)
        pid = 0
        for i in range(_NP):
            for j in range(i, _NP):
                A = att10[i] * att10[j]
                mny = jnp.maximum(y0s[i], y0s[j])
                mnx = jnp.maximum(x0s[i], x0s[j])
                mxy = jnp.minimum(y1s[i], y1s[j])
                mxx = jnp.minimum(x1s[i], x1s[j])
                ia = (jnp.maximum(mxy - mny, 0.0)
                      * jnp.maximum(mxx - mnx, 0.0))
                Dm = 1.0 - ia / (areas[i] + areas[j] - ia + 1e-12)
                plA0, _, _ = _pw(tab_v, A, 0)
                plD1, ipD, fpD = _pw(tab_v, Dm, 1)
                plD6 = _pw2(tab_v, ipD, fpD, 6)
                ad = 1.0 - jnp.abs(att10[i] - att10[j])
                sim, _, _ = _pw(tab_v, ad, 2)
                at_v[pid, :] = plA0 * plD1
                s_acc[i] = s_acc[i] + sim
                if i != j:
                    s_acc[j] = s_acc[j] + sim
                pd = jnp.abs(plD6 - 0.5)
                pd_sum = pd_sum + (pd if i == j else pd + pd)
                pid += 1

        inv_s = []
        for i in range(_NP):
            s_acc[i] = s_acc[i] * factor
            inv_s.append(1.0 / s_acc[i])

        modE = _splat_f(0.0)
        pid = 0
        for i in range(_NP):
            for j in range(i, _NP):
                sc = at_v[pid, :] * (inv_s[i] * inv_s[j])
                modE = modE + (sc if i == j else sc + sc)
                pid += 1
        for i in range(_NP):
            cn, _, _ = _pw(tab_v, att10[i] * att10[i], 0)
            modE = modE + cn * inv_s[i]

        a = modE + 1e-20
        zi = jnp.int32(0x5F3759DF) - (plsc.bitcast(a, jnp.int32) >> 1)
        y = plsc.bitcast(zi, jnp.float32)
        for _ in range(3):
            y = y * (1.5 - 0.5 * a * y * y)
        cval = a * y

        cc = jnp.clip(cval, 0.0, 10.0)
        ip = cc.astype(jnp.int32)
        fp = cc - ip.astype(jnp.float32)
        ip1 = jnp.minimum(ip + 1, _NP)
        pa_sum = _splat_f(0.0)
        for i in range(_NP):
            pl5, _, _ = _pw(tab_v, att10[i], 5)
            pa_sum = pa_sum + jnp.abs(pl5 - 0.5)
        gate, _, _ = _pw(tab_v, pa_sum / 10.0 + pd_sum / 100.0, 7)
        one = _splat_f(1.0)
        zero = _splat_f(0.0)
        obase = lane * (_NP + 1)
        for t in range(_NP + 1):
            o_t = (jnp.where(ip == t, one - fp, zero)
                   + jnp.where(ip1 == t, fp, zero))
            plsc.store_scatter(out_v, [obase + t], gate * o_t)
        pltpu.sync_copy(out_v, out_hbm.at[pl.ds(g * _L * (_NP + 1),
                                                _L * (_NP + 1))])
        return carry

    lax.fori_loop(0, gpw, group_body, 0)


def _pw_multi(xp, coef_ref, rows):
    res = [coef_ref[r, 0] * (xp + 1.0) for r in rows]
    for s in range(_D):
        t = xp if s == 0 else jnp.maximum(xp - float(s), 0.0)
        for j, r in enumerate(rows):
            res[j] = res[j] + coef_ref[r, s + 1] * t
    return res


def _tc_block(att_ref, boxes_ref, coef_ref, fac_ref, out_ref):
    att_all = att_ref[...]
    iota_n = jax.lax.broadcasted_iota(jnp.int32, (_N, _BLK), 0)

    work = att_all
    vals, coords = [], [[] for _ in range(4)]
    for _ in range(_NP):
        m = jnp.max(work, axis=0, keepdims=True)
        cand = jnp.where(work == m, iota_n, _N)
        sel = jnp.min(cand, axis=0, keepdims=True)
        onehot = iota_n == sel
        vals.append(m)
        ohf = onehot.astype(jnp.float32)
        for c in range(4):
            coords[c].append(
                jnp.sum(boxes_ref[c] * ohf, axis=0, keepdims=True))
        work = jnp.where(onehot, -jnp.inf, work)

    att_top = jnp.concatenate(vals, axis=0)
    y0, x0, y1, x1 = (jnp.concatenate(cs, axis=0) for cs in coords)

    e = jnp.exp(-jnp.abs(att_top))
    att = jnp.where(att_top >= 0.0, 1.0 / (1.0 + e), e / (1.0 + e))

    ai = att[:, None, :]
    aj = att[None, :, :]
    A = ai * aj

    h = jnp.maximum(y1 - y0, 0.0)
    w = jnp.maximum(x1 - x0, 0.0)
    areas = h * w
    mny = jnp.maximum(y0[:, None, :], y0[None, :, :])
    mnx = jnp.maximum(x0[:, None, :], x0[None, :, :])
    mxy = jnp.minimum(y1[:, None, :], y1[None, :, :])
    mxx = jnp.minimum(x1[:, None, :], x1[None, :, :])
    ia = jnp.maximum(mxy - mny, 0.0) * jnp.maximum(mxx - mnx, 0.0)
    iou = ia / (areas[:, None, :] + areas[None, :, :] - ia + 1e-12)
    Dm = 1.0 - iou

    (plA0,) = _pw_multi(A * 16.0, coef_ref, [0])
    plD1, plD6 = _pw_multi(Dm * 16.0, coef_ref, [1, 6])
    att_diff = jnp.abs(ai - aj)
    (sim,) = _pw_multi((1.0 - att_diff) * 16.0, coef_ref, [2])

    A_tilde = plA0 * plD1
    s_i = fac_ref[0, 0] * jnp.sum(sim, axis=1)
    score = A_tilde / (s_i[:, None, :] * s_i[None, :, :])
    (corr_num,) = _pw_multi(att * att * 16.0, coef_ref, [0])
    corr = corr_num / s_i
    mod_E = (jnp.sum(jnp.sum(score, axis=1), axis=0, keepdims=True)
             + jnp.sum(corr, axis=0, keepdims=True))
    c = jnp.sqrt(mod_E + 1e-20)

    cc = jnp.clip(c, 0.0, float(_NP))
    ip = cc.astype(jnp.int32)
    fp = cc - jnp.trunc(cc)
    iota11 = jax.lax.broadcasted_iota(jnp.int32, (_NP + 1, _BLK), 0)
    left = (iota11 == ip).astype(jnp.float32)
    right = (iota11 == jnp.minimum(ip + 1, _NP)).astype(jnp.float32)
    o = (1.0 - fp) * left + fp * right

    (pl5,) = _pw_multi(att * 16.0, coef_ref, [5])
    p_a = jnp.abs(pl5 - 0.5)
    pam = jnp.sum(p_a, axis=0, keepdims=True) / float(_NP)
    p_d = jnp.abs(plD6 - 0.5)
    pdm = jnp.sum(jnp.sum(p_d, axis=1) / float(_NP),
                  axis=0, keepdims=True) / float(_NP)
    (gate,) = _pw_multi((pam + pdm) * 16.0, coef_ref, [7])
    out_ref[...] = gate * o


@jax.jit
def kernel(boxes, attention, Ws):
    B = attention.shape[0]
    S = _SC_ROWS
    Btc = B - S

    aw = jnp.abs(Ws)
    nw = aw / jnp.sum(aw, axis=1, keepdims=True)
    cs = jnp.cumsum(nw, axis=1)
    fac = cs[2, 16] ** _NP

    if S:
        boxes_flat = boxes[:S].reshape(S, 4 * _N)
        tab = jnp.concatenate([cs.reshape(-1), nw.reshape(-1)])
        par = jnp.full((16,), fac, jnp.float32)
        mesh = plsc.VectorSubcoreMesh(core_axis_name="c",
                                      subcore_axis_name="s")
        sc_fn = pl.kernel(
            functools.partial(_sc_body, S // _L // _NW),
            out_type=jax.ShapeDtypeStruct((S * (_NP + 1),), jnp.float32),
            mesh=mesh,
            scratch_types=[
                pltpu.VMEM((_L, _N), jnp.float32),
                pltpu.VMEM((_L, 4 * _N), jnp.float32),
                pltpu.VMEM((272,), jnp.float32),
                pltpu.VMEM((16,), jnp.float32),
                pltpu.VMEM((56, _L), jnp.float32),
                pltpu.VMEM((_L * (_NP + 1),), jnp.float32),
            ],
            compiler_params=pltpu.CompilerParams(needs_layout_passes=False, use_tc_tiling_on_sc=True),
        )
        sc_out = sc_fn(attention[:S], boxes_flat, tab, par)

    att_t = attention[S:].T
    boxes_t = boxes[S:].transpose(1, 2, 0)
    coef = jnp.concatenate([nw[:, :1], nw[:, 1:] - nw[:, :-1]], axis=1)
    fac2 = fac.reshape(1, 1)
    tc_out = pl.pallas_call(
        _tc_block,
        grid=(Btc // _BLK,),
        in_specs=[
            pl.BlockSpec((_N, _BLK), lambda i: (0, i)),
            pl.BlockSpec((4, _N, _BLK), lambda i: (0, 0, i)),
            pl.BlockSpec(memory_space=pltpu.SMEM),
            pl.BlockSpec(memory_space=pltpu.SMEM),
        ],
        out_specs=pl.BlockSpec((_NP + 1, _BLK), lambda i: (0, i)),
        out_shape=jax.ShapeDtypeStruct((_NP + 1, Btc), jnp.float32),
    )(att_t, boxes_t, coef, fac2)

    if not S:
        return tc_out.T
    return jnp.concatenate([sc_out.reshape(S, _NP + 1), tc_out.T], axis=0)
